# baseline (speedup 1.0000x reference)
import functools

import jax
import jax.numpy as jnp
from jax.experimental import pallas as pl
from jax.experimental.pallas import tpu as pltpu
from jax.experimental.pallas import tpu_sc as plsc

_NUM_FEATURES = 128
_NUM_PIDS = 100000
_NUM_CQ = 5000
_OIM_SCALAR = 30.0
_MARGIN = 0.3
_BATCH = 1024

_BLK = 4000
_N_LUT_BLKS = _NUM_PIDS // _BLK
_GRID = _N_LUT_BLKS + 1


_SC_CORES = 2
_SC_SUBCORES = 16
_NW = _SC_CORES * _SC_SUBCORES
_BPW = _BATCH // _NW


@functools.lru_cache(maxsize=None)
def _make_sc_gather():
    @functools.partial(
        pl.kernel,
        mesh=plsc.VectorSubcoreMesh(core_axis_name="c", subcore_axis_name="s"),
        out_type=jax.ShapeDtypeStruct((_BATCH, _NUM_FEATURES), jnp.float32),
        scratch_types=[
            pltpu.VMEM((_BPW,), jnp.int32),
            pltpu.VMEM((_BPW, _NUM_FEATURES), jnp.float32),
            pltpu.SemaphoreType.DMA,
        ],
    )
    def _sc_gather(lut_hbm, idx_hbm, out_hbm, idx_v, rows_v, sem):
        wid = jax.lax.axis_index("s") * _SC_CORES + jax.lax.axis_index("c")
        base = wid * _BPW
        pltpu.sync_copy(idx_hbm.at[pl.ds(base, _BPW)], idx_v)
        pltpu.async_copy(lut_hbm.at[idx_v], rows_v, sem).wait()
        pltpu.sync_copy(rows_v, out_hbm.at[pl.ds(base, _BPW)])

    return _sc_gather


_LN2 = 0.6931471805599453

_BLK_F = 5000
_N_LUT_F = _NUM_PIDS // _BLK_F
_GRID_F = _N_LUT_F + 1
_SUB_F = 1000


_LOG2E = 1.4426950408889634


def _fused_body(x_ref, lut_ref, cq_ref, pf_ref, labc_ref, labr_ref,
                oim_ref, tri_ref, bad_ref, s_ref, xb_ref):
    step = pl.program_id(0)

    @pl.when(step == 0)
    def _():
        xb_ref[...] = (x_ref[...] * (_OIM_SCALAR * _LOG2E)).astype(
            jnp.bfloat16)

    xb = xb_ref[...]

    def _block_sum(e_ref, rows):
        acc = None
        for j in range(rows // _SUB_F):
            eb = e_ref[j * _SUB_F:(j + 1) * _SUB_F, :].astype(jnp.bfloat16)
            z = jax.lax.dot_general(
                xb, eb, (((1,), (1,)), ((), ())),
                preferred_element_type=jnp.float32)
            bs = jnp.sum(jnp.exp2(z), axis=1, keepdims=True)
            acc = bs if acc is None else acc + bs
        return acc

    @pl.when(step == 0)
    def _():
        s_ref[:, 0:1] = _block_sum(lut_ref, _BLK_F)

    @pl.when(jnp.logical_and(step > 0, step < _GRID_F - 1))
    def _():
        s_ref[:, 0:1] = s_ref[:, 0:1] + _block_sum(lut_ref, _BLK_F)

    @pl.when(step == _GRID_F - 1)
    def _():
        s = s_ref[:, 0:1] + _block_sum(cq_ref, _NUM_CQ)
        ok_row = (s > 0.0) & (s < 3.0e38)
        badv = jnp.max(jnp.where(ok_row, 0.0, 1.0), axis=0, keepdims=True)
        bad_ref[...] = jnp.broadcast_to(badv, (8, 128))
        lse = _LN2 * jnp.log2(s)
        _loss_math(x_ref, pf_ref, lse, labc_ref, labr_ref, oim_ref, tri_ref)


def _fused_call(x, lut, cq, pf, labc, labr):
    return pl.pallas_call(
        _fused_body,
        grid=(_GRID_F,),
        in_specs=[
            pl.BlockSpec((_BATCH, _NUM_FEATURES), lambda i: (0, 0)),
            pl.BlockSpec((_BLK_F, _NUM_FEATURES),
                         lambda i: (jnp.minimum(i, _N_LUT_F - 1), 0)),
            pl.BlockSpec((_NUM_CQ, _NUM_FEATURES), lambda i: (0, 0)),
            pl.BlockSpec((_BATCH, _NUM_FEATURES), lambda i: (0, 0)),
            pl.BlockSpec((2 * _BATCH, 128), lambda i: (0, 0)),
            pl.BlockSpec((8, 2 * _BATCH), lambda i: (0, 0)),
        ],
        out_specs=(
            pl.BlockSpec((_BATCH, 128), lambda i: (0, 0)),
            pl.BlockSpec((8, 128), lambda i: (0, 0)),
            pl.BlockSpec((8, 128), lambda i: (0, 0)),
        ),
        out_shape=(
            jax.ShapeDtypeStruct((_BATCH, 128), jnp.float32),
            jax.ShapeDtypeStruct((8, 128), jnp.float32),
            jax.ShapeDtypeStruct((8, 128), jnp.float32),
        ),
        scratch_shapes=[
            pltpu.VMEM((_BATCH, 128), jnp.float32),
            pltpu.VMEM((_BATCH, _NUM_FEATURES), jnp.bfloat16),
        ],
        compiler_params=pltpu.CompilerParams(
            dimension_semantics=("arbitrary",)),
    )(x, lut, cq, pf, labc, labr)


def _lse_body(x_ref, lut_ref, cq_ref, lse_ref, m_ref, s_ref):
    step = pl.program_id(0)
    xb = (x_ref[...] * (_OIM_SCALAR * _LOG2E)).astype(jnp.bfloat16)

    def _block_logits(e_ref):
        eb = e_ref[...].astype(jnp.bfloat16)
        return jax.lax.dot_general(
            xb, eb, (((1,), (1,)), ((), ())),
            preferred_element_type=jnp.float32)

    def _accum(e_ref):
        logits = _block_logits(e_ref)
        bm = jnp.max(logits, axis=1, keepdims=True)
        m_old = m_ref[:, 0:1]
        m_new = jnp.maximum(m_old, bm)
        bs = jnp.sum(jnp.exp2(logits - m_new), axis=1, keepdims=True)
        m_ref[:, 0:1] = m_new
        s_ref[:, 0:1] = s_ref[:, 0:1] * jnp.exp2(m_old - m_new) + bs

    @pl.when(step == 0)
    def _():
        logits = _block_logits(lut_ref)
        bm = jnp.max(logits, axis=1, keepdims=True)
        m_ref[:, 0:1] = bm
        s_ref[:, 0:1] = jnp.sum(jnp.exp2(logits - bm), axis=1, keepdims=True)

    @pl.when(jnp.logical_and(step > 0, step < _GRID - 1))
    def _():
        _accum(lut_ref)

    @pl.when(step == _GRID - 1)
    def _():
        _accum(cq_ref)
        lse = _LN2 * (m_ref[:, 0:1] + jnp.log2(s_ref[:, 0:1]))
        lse_ref[...] = jnp.broadcast_to(lse, (_BATCH, 128))


def _lse_call(xs_b16, lut, cq):
    return pl.pallas_call(
        _lse_body,
        grid=(_GRID,),
        in_specs=[
            pl.BlockSpec((_BATCH, _NUM_FEATURES), lambda i: (0, 0)),
            pl.BlockSpec((_BLK, _NUM_FEATURES),
                         lambda i: (jnp.minimum(i, _N_LUT_BLKS - 1), 0)),
            pl.BlockSpec((_NUM_CQ, _NUM_FEATURES), lambda i: (0, 0)),
        ],
        out_specs=pl.BlockSpec((_BATCH, 128), lambda i: (0, 0)),
        out_shape=jax.ShapeDtypeStruct((_BATCH, 128), jnp.float32),
        scratch_shapes=[
            pltpu.VMEM((_BATCH, 128), jnp.float32),
            pltpu.VMEM((_BATCH, 128), jnp.float32),
        ],
        compiler_params=pltpu.CompilerParams(
            dimension_semantics=("arbitrary",)),
    )(xs_b16, lut, cq)


_BIG = 1e30
_BIG_CHK = 1e29


def _loss_math(x_ref, pf_ref, lse, labc_ref, labr_ref, oim_ref, tri_ref):
    x = x_ref[...]
    pf = pf_ref[...]

    logit_lab = _OIM_SCALAR * jnp.sum(x * pf, axis=1, keepdims=True)
    log_p = logit_lab - lse
    p = jnp.exp(log_p)
    focal = (1.0 - p) ** 2 * log_p
    oim_ref[...] = jnp.broadcast_to(-focal, (_BATCH, 128))

    feats = jnp.concatenate([x, pf], axis=0).astype(jnp.bfloat16)
    feats2 = (feats.astype(jnp.float32) ** 2).astype(jnp.bfloat16)
    sq_c = jnp.sum(feats2.astype(jnp.float32), axis=1, keepdims=True)
    ones_row = jnp.ones((1, _NUM_FEATURES), jnp.bfloat16)
    sq_r = jax.lax.dot_general(
        ones_row, feats2, (((1,), (1,)), ((), ())),
        preferred_element_type=jnp.float32)
    g = jax.lax.dot_general(
        feats, feats, (((1,), (1,)), ((), ())),
        preferred_element_type=jnp.float32)
    d2 = (sq_c + sq_r) - 2.0 * g

    lab_c = labc_ref[:, 0:1]
    lab_r = labr_ref[0:1, :]
    same = lab_c == lab_r
    same_f = same.astype(jnp.bfloat16)
    n2 = 2 * _BATCH
    ones_col = jnp.ones((n2, 1), jnp.bfloat16)
    cnt_c = jax.lax.dot_general(
        same_f, ones_col, (((1,), (0,)), ((), ())),
        preferred_element_type=jnp.float32)
    cnt_r = jax.lax.dot_general(
        jnp.ones((1, n2), jnp.bfloat16), same_f, (((1,), (0,)), ((), ())),
        preferred_element_type=jnp.float32)
    keep_c = (lab_c >= 0) & (cnt_c >= 2.0)
    keep_r = (lab_r >= 0) & (cnt_r >= 2.0)

    ap2 = jnp.max(jnp.where(same, d2, -_BIG), axis=1, keepdims=True)
    pen_r = jnp.where(keep_r, 0.0, _BIG)
    an2 = jnp.min(jnp.where(same, _BIG, d2 + pen_r), axis=1, keepdims=True)
    dist_ap = jnp.sqrt(jnp.maximum(ap2, 1e-12))
    dist_an = jnp.sqrt(jnp.maximum(an2, 1e-12))
    anchor_ok = keep_c & (ap2 > -_BIG_CHK) & (an2 < _BIG_CHK)
    losses = jnp.maximum(dist_ap - dist_an + _MARGIN, 0.0)
    losses = jnp.where(anchor_ok, losses, 0.0)
    ok_f = anchor_ok.astype(jnp.float32)
    denom = jnp.maximum(jnp.sum(ok_f, axis=0, keepdims=True), 1.0)
    tri = jnp.sum(losses, axis=0, keepdims=True) / denom
    tri_ref[...] = jnp.broadcast_to(tri, (8, 128))


def _loss_body(x_ref, pf_ref, lse_ref, labc_ref, labr_ref, oim_ref, tri_ref):
    _loss_math(x_ref, pf_ref, lse_ref[:, 0:1], labc_ref, labr_ref,
               oim_ref, tri_ref)


def _loss_call(x, pf, lse_b, labc, labr):
    return pl.pallas_call(
        _loss_body,
        out_shape=(
            jax.ShapeDtypeStruct((_BATCH, 128), jnp.float32),
            jax.ShapeDtypeStruct((8, 128), jnp.float32),
        ),
    )(x, pf, lse_b, labc, labr)


def kernel(inputs, roi_label, lut, cq, cq_omega):
    label = roi_label.astype(jnp.int32)
    pos_feats = _make_sc_gather()(lut, label)

    pos_pids = jnp.where(label > 0, label,
                         -2 - jnp.arange(_BATCH, dtype=jnp.int32))
    labels2 = jnp.concatenate([label, pos_pids]).astype(jnp.float32)
    labc = jnp.broadcast_to(labels2[:, None], (2 * _BATCH, 128))
    labr = jnp.broadcast_to(labels2[None, :], (8, 2 * _BATCH))

    oim_f, tri_f, badf = _fused_call(inputs, lut, cq, pos_feats, labc, labr)
    bad = badf[0, 0] > 0.5

    def _slow_path():
        lse_s = _lse_call(inputs, lut, cq)
        return _loss_call(inputs, pos_feats, lse_s, labc, labr)

    oim, tri = jax.lax.cond(bad, _slow_path, lambda: (oim_f, tri_f))
    return oim[:, 0], tri[0, 0]


# speedup vs baseline: 1.1329x; 1.1329x over previous
import functools

import jax
import jax.numpy as jnp
from jax.experimental import pallas as pl
from jax.experimental.pallas import tpu as pltpu
from jax.experimental.pallas import tpu_sc as plsc

_NUM_FEATURES = 128
_NUM_PIDS = 100000
_NUM_CQ = 5000
_OIM_SCALAR = 30.0
_MARGIN = 0.3
_BATCH = 1024

_LN2 = 0.6931471805599453
_LOG2E = 1.4426950408889634


_SC_CORES = 2
_SC_SUBCORES = 16
_NW = _SC_CORES * _SC_SUBCORES
_BPW = _BATCH // _NW


@functools.lru_cache(maxsize=None)
def _make_sc_gather():
    @functools.partial(
        pl.kernel,
        mesh=plsc.VectorSubcoreMesh(core_axis_name="c", subcore_axis_name="s"),
        out_type=jax.ShapeDtypeStruct((_BATCH, _NUM_FEATURES), jnp.float32),
        scratch_types=[
            pltpu.VMEM((_BPW,), jnp.int32),
            pltpu.VMEM((_BPW, _NUM_FEATURES), jnp.float32),
            pltpu.SemaphoreType.DMA,
        ],
    )
    def _sc_gather(lut_hbm, idx_hbm, out_hbm, idx_v, rows_v, sem):
        wid = jax.lax.axis_index("s") * _SC_CORES + jax.lax.axis_index("c")
        base = wid * _BPW
        pltpu.sync_copy(idx_hbm.at[pl.ds(base, _BPW)], idx_v)
        pltpu.async_copy(lut_hbm.at[idx_v], rows_v, sem).wait()
        pltpu.sync_copy(rows_v, out_hbm.at[pl.ds(base, _BPW)])

    return _sc_gather


_BLK_F = 5000
_N_LUT_F = _NUM_PIDS // _BLK_F
_GRID_F = _N_LUT_F + 1
_SUB_F = 1000


def _lse_fast_body(x_ref, lut_ref, cq_ref, lse_ref, s_ref):
    step = pl.program_id(0)
    xb = x_ref[...]

    def _block_sum(e_ref, rows):
        acc = None
        for j in range(rows // _SUB_F):
            eb = e_ref[j * _SUB_F:(j + 1) * _SUB_F, :].astype(jnp.bfloat16)
            z = jax.lax.dot_general(
                xb, eb, (((1,), (1,)), ((), ())),
                preferred_element_type=jnp.float32)
            bs = jnp.sum(jnp.exp2(z), axis=1, keepdims=True)
            acc = bs if acc is None else acc + bs
        return acc

    @pl.when(step == 0)
    def _():
        s_ref[:, 0:1] = _block_sum(lut_ref, _BLK_F)

    @pl.when(jnp.logical_and(step > 0, step < _GRID_F - 1))
    def _():
        s_ref[:, 0:1] = s_ref[:, 0:1] + _block_sum(lut_ref, _BLK_F)

    @pl.when(step == _GRID_F - 1)
    def _():
        s = s_ref[:, 0:1] + _block_sum(cq_ref, _NUM_CQ)
        lse = _LN2 * jnp.log2(s)
        lse_ref[...] = jnp.broadcast_to(lse, (_BATCH, 128))


def _lse_fast_call(xs_b16, lut, cq):
    return pl.pallas_call(
        _lse_fast_body,
        grid=(_GRID_F,),
        in_specs=[
            pl.BlockSpec((_BATCH, _NUM_FEATURES), lambda i: (0, 0)),
            pl.BlockSpec((_BLK_F, _NUM_FEATURES),
                         lambda i: (jnp.minimum(i, _N_LUT_F - 1), 0)),
            pl.BlockSpec((_NUM_CQ, _NUM_FEATURES), lambda i: (0, 0)),
        ],
        out_specs=pl.BlockSpec((_BATCH, 128), lambda i: (0, 0)),
        out_shape=jax.ShapeDtypeStruct((_BATCH, 128), jnp.float32),
        scratch_shapes=[pltpu.VMEM((_BATCH, 128), jnp.float32)],
        compiler_params=pltpu.CompilerParams(
            dimension_semantics=("arbitrary",)),
    )(xs_b16, lut, cq)


_BLK = 4000
_N_LUT_BLKS = _NUM_PIDS // _BLK
_GRID = _N_LUT_BLKS + 1


def _lse_body(x_ref, lut_ref, cq_ref, lse_ref, m_ref, s_ref):
    step = pl.program_id(0)
    xb = x_ref[...]

    def _block_logits(e_ref):
        eb = e_ref[...].astype(jnp.bfloat16)
        return jax.lax.dot_general(
            xb, eb, (((1,), (1,)), ((), ())),
            preferred_element_type=jnp.float32)

    def _accum(e_ref):
        logits = _block_logits(e_ref)
        bm = jnp.max(logits, axis=1, keepdims=True)
        m_old = m_ref[:, 0:1]
        m_new = jnp.maximum(m_old, bm)
        bs = jnp.sum(jnp.exp2(logits - m_new), axis=1, keepdims=True)
        m_ref[:, 0:1] = m_new
        s_ref[:, 0:1] = s_ref[:, 0:1] * jnp.exp2(m_old - m_new) + bs

    @pl.when(step == 0)
    def _():
        logits = _block_logits(lut_ref)
        bm = jnp.max(logits, axis=1, keepdims=True)
        m_ref[:, 0:1] = bm
        s_ref[:, 0:1] = jnp.sum(jnp.exp2(logits - bm), axis=1, keepdims=True)

    @pl.when(jnp.logical_and(step > 0, step < _GRID - 1))
    def _():
        _accum(lut_ref)

    @pl.when(step == _GRID - 1)
    def _():
        _accum(cq_ref)
        lse = _LN2 * (m_ref[:, 0:1] + jnp.log2(s_ref[:, 0:1]))
        lse_ref[...] = jnp.broadcast_to(lse, (_BATCH, 128))


def _lse_call(xs_b16, lut, cq):
    return pl.pallas_call(
        _lse_body,
        grid=(_GRID,),
        in_specs=[
            pl.BlockSpec((_BATCH, _NUM_FEATURES), lambda i: (0, 0)),
            pl.BlockSpec((_BLK, _NUM_FEATURES),
                         lambda i: (jnp.minimum(i, _N_LUT_BLKS - 1), 0)),
            pl.BlockSpec((_NUM_CQ, _NUM_FEATURES), lambda i: (0, 0)),
        ],
        out_specs=pl.BlockSpec((_BATCH, 128), lambda i: (0, 0)),
        out_shape=jax.ShapeDtypeStruct((_BATCH, 128), jnp.float32),
        scratch_shapes=[
            pltpu.VMEM((_BATCH, 128), jnp.float32),
            pltpu.VMEM((_BATCH, 128), jnp.float32),
        ],
        compiler_params=pltpu.CompilerParams(
            dimension_semantics=("arbitrary",)),
    )(xs_b16, lut, cq)


_BIG = 1e30
_BIG_CHK = 1e29


def _loss_body(x_ref, pf_ref, lse_ref, labc_ref, labr_ref, oim_ref, tri_ref):
    x = x_ref[...]
    pf = pf_ref[...]
    lse = lse_ref[:, 0:1]

    logit_lab = _OIM_SCALAR * jnp.sum(x * pf, axis=1, keepdims=True)
    log_p = logit_lab - lse
    p = jnp.exp(log_p)
    focal = (1.0 - p) ** 2 * log_p
    oim_ref[...] = jnp.broadcast_to(-focal, (_BATCH, 128))

    feats = jnp.concatenate([x, pf], axis=0)
    feats2 = feats * feats
    sq_c = jnp.sum(feats2, axis=1, keepdims=True)
    ones_row = jnp.ones((1, _NUM_FEATURES), jnp.float32)
    sq_r = jax.lax.dot_general(
        ones_row, feats2, (((1,), (1,)), ((), ())),
        preferred_element_type=jnp.float32)
    g = jax.lax.dot_general(
        feats, feats, (((1,), (1,)), ((), ())),
        preferred_element_type=jnp.float32)
    d2 = (sq_c + sq_r) - 2.0 * g

    lab_c = labc_ref[:, 0:1]
    lab_r = labr_ref[0:1, :]
    same = lab_c == lab_r
    same_f = same.astype(jnp.bfloat16)
    n2 = 2 * _BATCH
    ones_col = jnp.ones((n2, 1), jnp.bfloat16)
    cnt_c = jax.lax.dot_general(
        same_f, ones_col, (((1,), (0,)), ((), ())),
        preferred_element_type=jnp.float32)
    cnt_r = jax.lax.dot_general(
        jnp.ones((1, n2), jnp.bfloat16), same_f, (((1,), (0,)), ((), ())),
        preferred_element_type=jnp.float32)
    keep_c = (lab_c >= 0) & (cnt_c >= 2.0)
    keep_r = (lab_r >= 0) & (cnt_r >= 2.0)

    ap2 = jnp.max(jnp.where(same, d2, -_BIG), axis=1, keepdims=True)
    pen_r = jnp.where(keep_r, 0.0, _BIG)
    an2 = jnp.min(jnp.where(same, _BIG, d2 + pen_r), axis=1, keepdims=True)
    dist_ap = jnp.sqrt(jnp.maximum(ap2, 1e-12))
    dist_an = jnp.sqrt(jnp.maximum(an2, 1e-12))
    anchor_ok = keep_c & (ap2 > -_BIG_CHK) & (an2 < _BIG_CHK)
    losses = jnp.maximum(dist_ap - dist_an + _MARGIN, 0.0)
    losses = jnp.where(anchor_ok, losses, 0.0)
    ok_f = anchor_ok.astype(jnp.float32)
    denom = jnp.maximum(jnp.sum(ok_f, axis=0, keepdims=True), 1.0)
    tri = jnp.sum(losses, axis=0, keepdims=True) / denom
    tri_ref[...] = jnp.broadcast_to(tri, (8, 128))


def _loss_call(x, pf, lse_b, labc, labr):
    return pl.pallas_call(
        _loss_body,
        out_shape=(
            jax.ShapeDtypeStruct((_BATCH, 128), jnp.float32),
            jax.ShapeDtypeStruct((8, 128), jnp.float32),
        ),
    )(x, pf, lse_b, labc, labr)


def kernel(inputs, roi_label, lut, cq, cq_omega):
    label = roi_label.astype(jnp.int32)
    pos_feats = _make_sc_gather()(lut, label)

    xs_b16 = (inputs * (_OIM_SCALAR * _LOG2E)).astype(jnp.bfloat16)
    lse_fast = _lse_fast_call(xs_b16, lut, cq)
    bad = jnp.logical_not(jnp.all(jnp.isfinite(lse_fast[:, 0])))
    lse_b = jax.lax.cond(
        bad, lambda: _lse_call(xs_b16, lut, cq), lambda: lse_fast)

    pos_pids = jnp.where(label > 0, label,
                         -2 - jnp.arange(_BATCH, dtype=jnp.int32))
    labels2 = jnp.concatenate([label, pos_pids]).astype(jnp.float32)
    labc = jnp.broadcast_to(labels2[:, None], (2 * _BATCH, 128))
    labr = jnp.broadcast_to(labels2[None, :], (8, 2 * _BATCH))

    oim, tri = _loss_call(inputs, pos_feats, lse_b, labc, labr)
    return oim[:, 0], tri[0, 0]


# speedup vs baseline: 1.1601x; 1.0240x over previous
import functools

import jax
import jax.numpy as jnp
from jax.experimental import pallas as pl
from jax.experimental.pallas import tpu as pltpu
from jax.experimental.pallas import tpu_sc as plsc

_NUM_FEATURES = 128
_NUM_PIDS = 100000
_NUM_CQ = 5000
_OIM_SCALAR = 30.0
_MARGIN = 0.3
_BATCH = 1024

_LN2 = 0.6931471805599453
_LOG2E = 1.4426950408889634


_SC_CORES = 2
_SC_SUBCORES = 16
_NW = _SC_CORES * _SC_SUBCORES
_BPW = _BATCH // _NW


@functools.lru_cache(maxsize=None)
def _make_sc_gather():
    @functools.partial(
        pl.kernel,
        mesh=plsc.VectorSubcoreMesh(core_axis_name="c", subcore_axis_name="s"),
        out_type=jax.ShapeDtypeStruct((_BATCH, _NUM_FEATURES), jnp.float32),
        scratch_types=[
            pltpu.VMEM((_BPW,), jnp.int32),
            pltpu.VMEM((_BPW, _NUM_FEATURES), jnp.float32),
            pltpu.SemaphoreType.DMA,
        ],
    )
    def _sc_gather(lut_hbm, idx_hbm, out_hbm, idx_v, rows_v, sem):
        wid = jax.lax.axis_index("s") * _SC_CORES + jax.lax.axis_index("c")
        base = wid * _BPW
        pltpu.sync_copy(idx_hbm.at[pl.ds(base, _BPW)], idx_v)
        pltpu.async_copy(lut_hbm.at[idx_v], rows_v, sem).wait()
        pltpu.sync_copy(rows_v, out_hbm.at[pl.ds(base, _BPW)])

    return _sc_gather


_BLK_F = 5000
_N_LUT_F = _NUM_PIDS // _BLK_F
_GRID_F = _N_LUT_F + 1
_SUB_F = 1000


def _lse_fast_body(x_ref, lut_ref, cq_ref, lse_ref, s_ref):
    step = pl.program_id(0)
    xb = x_ref[...]

    def _block_sum(e_ref, rows):
        acc = None
        for j in range(rows // _SUB_F):
            eb = e_ref[j * _SUB_F:(j + 1) * _SUB_F, :].astype(jnp.bfloat16)
            z = jax.lax.dot_general(
                xb, eb, (((1,), (1,)), ((), ())),
                preferred_element_type=jnp.float32).astype(jnp.bfloat16)
            bs = jnp.sum(jnp.exp2(z).astype(jnp.float32), axis=1,
                         keepdims=True)
            acc = bs if acc is None else acc + bs
        return acc

    @pl.when(step == 0)
    def _():
        s_ref[:, 0:1] = _block_sum(lut_ref, _BLK_F)

    @pl.when(jnp.logical_and(step > 0, step < _GRID_F - 1))
    def _():
        s_ref[:, 0:1] = s_ref[:, 0:1] + _block_sum(lut_ref, _BLK_F)

    @pl.when(step == _GRID_F - 1)
    def _():
        s = s_ref[:, 0:1] + _block_sum(cq_ref, _NUM_CQ)
        lse = _LN2 * jnp.log2(s)
        lse_ref[...] = jnp.broadcast_to(lse, (_BATCH, 128))


def _lse_fast_call(xs_b16, lut, cq):
    return pl.pallas_call(
        _lse_fast_body,
        grid=(_GRID_F,),
        in_specs=[
            pl.BlockSpec((_BATCH, _NUM_FEATURES), lambda i: (0, 0)),
            pl.BlockSpec((_BLK_F, _NUM_FEATURES),
                         lambda i: (jnp.minimum(i, _N_LUT_F - 1), 0)),
            pl.BlockSpec((_NUM_CQ, _NUM_FEATURES), lambda i: (0, 0)),
        ],
        out_specs=pl.BlockSpec((_BATCH, 128), lambda i: (0, 0)),
        out_shape=jax.ShapeDtypeStruct((_BATCH, 128), jnp.float32),
        scratch_shapes=[pltpu.VMEM((_BATCH, 128), jnp.float32)],
        compiler_params=pltpu.CompilerParams(
            dimension_semantics=("arbitrary",)),
    )(xs_b16, lut, cq)


_BLK = 4000
_N_LUT_BLKS = _NUM_PIDS // _BLK
_GRID = _N_LUT_BLKS + 1


def _lse_body(x_ref, lut_ref, cq_ref, lse_ref, m_ref, s_ref):
    step = pl.program_id(0)
    xb = x_ref[...]

    def _block_logits(e_ref):
        eb = e_ref[...].astype(jnp.bfloat16)
        return jax.lax.dot_general(
            xb, eb, (((1,), (1,)), ((), ())),
            preferred_element_type=jnp.float32)

    def _accum(e_ref):
        logits = _block_logits(e_ref)
        bm = jnp.max(logits, axis=1, keepdims=True)
        m_old = m_ref[:, 0:1]
        m_new = jnp.maximum(m_old, bm)
        bs = jnp.sum(jnp.exp2(logits - m_new), axis=1, keepdims=True)
        m_ref[:, 0:1] = m_new
        s_ref[:, 0:1] = s_ref[:, 0:1] * jnp.exp2(m_old - m_new) + bs

    @pl.when(step == 0)
    def _():
        logits = _block_logits(lut_ref)
        bm = jnp.max(logits, axis=1, keepdims=True)
        m_ref[:, 0:1] = bm
        s_ref[:, 0:1] = jnp.sum(jnp.exp2(logits - bm), axis=1, keepdims=True)

    @pl.when(jnp.logical_and(step > 0, step < _GRID - 1))
    def _():
        _accum(lut_ref)

    @pl.when(step == _GRID - 1)
    def _():
        _accum(cq_ref)
        lse = _LN2 * (m_ref[:, 0:1] + jnp.log2(s_ref[:, 0:1]))
        lse_ref[...] = jnp.broadcast_to(lse, (_BATCH, 128))


def _lse_call(xs_b16, lut, cq):
    return pl.pallas_call(
        _lse_body,
        grid=(_GRID,),
        in_specs=[
            pl.BlockSpec((_BATCH, _NUM_FEATURES), lambda i: (0, 0)),
            pl.BlockSpec((_BLK, _NUM_FEATURES),
                         lambda i: (jnp.minimum(i, _N_LUT_BLKS - 1), 0)),
            pl.BlockSpec((_NUM_CQ, _NUM_FEATURES), lambda i: (0, 0)),
        ],
        out_specs=pl.BlockSpec((_BATCH, 128), lambda i: (0, 0)),
        out_shape=jax.ShapeDtypeStruct((_BATCH, 128), jnp.float32),
        scratch_shapes=[
            pltpu.VMEM((_BATCH, 128), jnp.float32),
            pltpu.VMEM((_BATCH, 128), jnp.float32),
        ],
        compiler_params=pltpu.CompilerParams(
            dimension_semantics=("arbitrary",)),
    )(xs_b16, lut, cq)


_BIG = 1e30
_BIG_CHK = 1e29


def _loss_body(x_ref, pf_ref, lse_ref, labc_ref, labr_ref, oim_ref, tri_ref):
    x = x_ref[...]
    pf = pf_ref[...]
    lse = lse_ref[:, 0:1]

    logit_lab = _OIM_SCALAR * jnp.sum(x * pf, axis=1, keepdims=True)
    log_p = logit_lab - lse
    p = jnp.exp(log_p)
    focal = (1.0 - p) ** 2 * log_p
    oim_ref[...] = jnp.broadcast_to(-focal, (_BATCH, 128))

    feats = jnp.concatenate([x, pf], axis=0)
    feats2 = feats * feats
    sq_c = jnp.sum(feats2, axis=1, keepdims=True)
    ones_row = jnp.ones((1, _NUM_FEATURES), jnp.float32)
    sq_r = jax.lax.dot_general(
        ones_row, feats2, (((1,), (1,)), ((), ())),
        preferred_element_type=jnp.float32)
    g = jax.lax.dot_general(
        feats, feats, (((1,), (1,)), ((), ())),
        preferred_element_type=jnp.float32)
    d2 = (sq_c + sq_r) - 2.0 * g

    lab_c = labc_ref[:, 0:1]
    lab_r = labr_ref[0:1, :]
    same = lab_c == lab_r
    same_f = same.astype(jnp.bfloat16)
    n2 = 2 * _BATCH
    ones_col = jnp.ones((n2, 1), jnp.bfloat16)
    cnt_c = jax.lax.dot_general(
        same_f, ones_col, (((1,), (0,)), ((), ())),
        preferred_element_type=jnp.float32)
    cnt_r = jax.lax.dot_general(
        jnp.ones((1, n2), jnp.bfloat16), same_f, (((1,), (0,)), ((), ())),
        preferred_element_type=jnp.float32)
    keep_c = (lab_c >= 0) & (cnt_c >= 2.0)
    keep_r = (lab_r >= 0) & (cnt_r >= 2.0)

    ap2 = jnp.max(jnp.where(same, d2, -_BIG), axis=1, keepdims=True)
    pen_r = jnp.where(keep_r, 0.0, _BIG)
    an2 = jnp.min(jnp.where(same, _BIG, d2 + pen_r), axis=1, keepdims=True)
    dist_ap = jnp.sqrt(jnp.maximum(ap2, 1e-12))
    dist_an = jnp.sqrt(jnp.maximum(an2, 1e-12))
    anchor_ok = keep_c & (ap2 > -_BIG_CHK) & (an2 < _BIG_CHK)
    losses = jnp.maximum(dist_ap - dist_an + _MARGIN, 0.0)
    losses = jnp.where(anchor_ok, losses, 0.0)
    ok_f = anchor_ok.astype(jnp.float32)
    denom = jnp.maximum(jnp.sum(ok_f, axis=0, keepdims=True), 1.0)
    tri = jnp.sum(losses, axis=0, keepdims=True) / denom
    tri_ref[...] = jnp.broadcast_to(tri, (8, 128))


def _loss_call(x, pf, lse_b, labc, labr):
    return pl.pallas_call(
        _loss_body,
        out_shape=(
            jax.ShapeDtypeStruct((_BATCH, 128), jnp.float32),
            jax.ShapeDtypeStruct((8, 128), jnp.float32),
        ),
    )(x, pf, lse_b, labc, labr)


def kernel(inputs, roi_label, lut, cq, cq_omega):
    label = roi_label.astype(jnp.int32)
    pos_feats = _make_sc_gather()(lut, label)

    xs_b16 = (inputs * (_OIM_SCALAR * _LOG2E)).astype(jnp.bfloat16)
    lse_fast = _lse_fast_call(xs_b16, lut, cq)
    bad = jnp.logical_not(jnp.all(jnp.isfinite(lse_fast[:, 0])))
    lse_b = jax.lax.cond(
        bad, lambda: _lse_call(xs_b16, lut, cq), lambda: lse_fast)

    pos_pids = jnp.where(label > 0, label,
                         -2 - jnp.arange(_BATCH, dtype=jnp.int32))
    labels2 = jnp.concatenate([label, pos_pids]).astype(jnp.float32)
    labc = jnp.broadcast_to(labels2[:, None], (2 * _BATCH, 128))
    labr = jnp.broadcast_to(labels2[None, :], (8, 2 * _BATCH))

    oim, tri = _loss_call(inputs, pos_feats, lse_b, labc, labr)
    return oim[:, 0], tri[0, 0]
